# baseline (speedup 1.0000x reference)
import functools
import jax
import jax.numpy as jnp
from jax import lax
from jax.experimental import pallas as pl
from jax.experimental.pallas import tpu as pltpu
from jax.experimental.pallas import tpu_sc as plsc

SMOOTHING = 0.1
IGNORE_INDEX = 0


def _make_sc_gather(n_rows, n_classes):
    info = plsc.get_sparse_core_info()
    nc, ns = info.num_cores, info.num_subcores
    nw = nc * ns
    b_per_w = n_rows // nw
    assert n_rows % nw == 0 and b_per_w % 16 == 0
    mesh = plsc.VectorSubcoreMesh(core_axis_name="c", subcore_axis_name="s")

    @functools.partial(
        pl.kernel, mesh=mesh,
        out_type=jax.ShapeDtypeStruct((n_rows,), jnp.float32),
        scratch_types=[
            pltpu.VMEM((b_per_w,), jnp.int32),
            pltpu.VMEM((b_per_w,), jnp.int32),
            pltpu.VMEM((b_per_w,), jnp.float32),
            pltpu.SemaphoreType.DMA,
        ],
    )
    def sc_gather(pred_flat_hbm, tgt_hbm, out_hbm, tgt_v, idx_v, val_v, sem):
        wid = lax.axis_index("s") * nc + lax.axis_index("c")
        base = wid * b_per_w
        pltpu.sync_copy(tgt_hbm.at[pl.ds(base, b_per_w)], tgt_v)
        for j in range(b_per_w // 16):
            row = lax.iota(jnp.int32, 16) + (base + j * 16)
            t = tgt_v[pl.ds(j * 16, 16)]
            idx_v[pl.ds(j * 16, 16)] = row * n_classes + t
        pltpu.async_copy(pred_flat_hbm.at[idx_v], val_v, sem).wait()
        pltpu.sync_copy(val_v, out_hbm.at[pl.ds(base, b_per_w)])

    return sc_gather


def _loss_body(pred_ref, tgt_ref, tval_ref, out_ref, m_ref, s_ref, psum_ref,
               p0_ref, *, n_col_blocks, blk_cols, n_classes):
    cb = pl.program_id(0)
    x = pred_ref[...]
    rows = x.shape[0]
    last = n_col_blocks - 1

    @pl.when(cb == 0)
    def _init():
        bm = jnp.max(x, axis=1, keepdims=True)
        m_ref[...] = bm
        s_ref[...] = jnp.sum(jnp.exp(x - bm), axis=1, keepdims=True)
        psum_ref[...] = jnp.sum(x, axis=1, keepdims=True)
        p0_ref[...] = x[:, 0:1]

    @pl.when((cb != 0) & (cb != last))
    def _acc():
        bm = jnp.max(x, axis=1, keepdims=True)
        m_old = m_ref[...]
        m_new = jnp.maximum(m_old, bm)
        s_ref[...] = (s_ref[...] * jnp.exp(m_old - m_new)
                      + jnp.sum(jnp.exp(x - m_new), axis=1, keepdims=True))
        m_ref[...] = m_new
        psum_ref[...] += jnp.sum(x, axis=1, keepdims=True)

    @pl.when(cb == last)
    def _fin():
        col = (jax.lax.broadcasted_iota(jnp.int32, (1, blk_cols), 1)
               + cb * blk_cols)
        valid = col < n_classes
        xm = jnp.where(valid, x, -jnp.inf)
        bm = jnp.max(xm, axis=1, keepdims=True)
        m_old = m_ref[...]
        m_new = jnp.maximum(m_old, bm)
        s = (s_ref[...] * jnp.exp(m_old - m_new)
             + jnp.sum(jnp.exp(xm - m_new), axis=1, keepdims=True))
        psum = psum_ref[...] + jnp.sum(jnp.where(valid, x, 0.0), axis=1,
                                       keepdims=True)

        eps = SMOOTHING / (n_classes - 1)
        conf = 1.0 - SMOOTHING
        tgt = tgt_ref[...]
        lse = m_new + jnp.log(s)
        s_logp = psum - n_classes * lse
        tlp = tval_ref[...] - lse
        zlp = p0_ref[...] - lse
        loss = -(eps * (s_logp - tlp - zlp) + conf * tlp)
        loss = jnp.where(tgt == IGNORE_INDEX, 0.0, loss)
        out_ref[...] = jnp.sum(loss, axis=0, keepdims=True) / rows


def kernel(pred, target):
    n, c = pred.shape
    tgt32 = target.astype(jnp.int32)
    tval = _make_sc_gather(n, c)(pred.reshape(-1), tgt32)

    blk_cols = 2048
    n_col_blocks = pl.cdiv(c, blk_cols)

    out = pl.pallas_call(
        functools.partial(_loss_body, n_col_blocks=n_col_blocks,
                          blk_cols=blk_cols, n_classes=c),
        grid=(n_col_blocks,),
        in_specs=[
            pl.BlockSpec((n, blk_cols), lambda cb: (0, cb)),
            pl.BlockSpec((n, 1), lambda cb: (0, 0)),
            pl.BlockSpec((n, 1), lambda cb: (0, 0)),
        ],
        out_specs=pl.BlockSpec((1, 1), lambda cb: (0, 0)),
        out_shape=jax.ShapeDtypeStruct((1, 1), jnp.float32),
        scratch_shapes=[
            pltpu.VMEM((n, 1), jnp.float32),
            pltpu.VMEM((n, 1), jnp.float32),
            pltpu.VMEM((n, 1), jnp.float32),
            pltpu.VMEM((n, 1), jnp.float32),
        ],
    )(pred, tgt32.reshape(n, 1), tval.reshape(n, 1))
    return out[0, 0]
